# baseline (speedup 1.0000x reference)
import functools

import jax
import jax.numpy as jnp
from jax import lax
from jax.experimental import pallas as pl
from jax.experimental.pallas import tpu as pltpu
from jax.experimental.pallas import tpu_sc as plsc

_B, _S, _D = 2, 4096, 1024
_E, _K, _FF = 8, 2, 4096
_T = _B * _S

_TS = 256
_N = _T * _K
_CAP = _N + _E * _TS
_NT = _CAP // _TS

_NC, _NS, _L = 2, 16, 16
_NW = _NC * _NS
_BPW = _T // _NW
_CC = 32
_NCH = _BPW // _CC


def _group_ffn_kernel(eot_ref, xs_ref, w_ref, gate_ref, up_ref, down_ref,
                      ys_ref):
    xt = xs_ref[...]
    g = gate_ref[0]
    u = up_ref[0]
    dn = down_ref[0]
    a = lax.dot_general(xt, g,
                        dimension_numbers=(((1,), (1,)), ((), ())),
                        preferred_element_type=jnp.float32)
    bb = lax.dot_general(xt, u,
                         dimension_numbers=(((1,), (1,)), ((), ())),
                         preferred_element_type=jnp.float32)
    h = (a * lax.logistic(a) * bb).astype(jnp.bfloat16)
    y = lax.dot_general(h, dn,
                        dimension_numbers=(((1,), (1,)), ((), ())),
                        preferred_element_type=jnp.float32)
    ys_ref[...] = y * w_ref[...]


def _sc_combine_body(ysw_hbm, p0_hbm, p1_hbm, out_hbm,
                     idx0_v, idx1_v, rows0_v, rows1_v, sem):
    wid = lax.axis_index("s") * _NC + lax.axis_index("c")
    base = wid * _BPW
    for c in range(_NCH):
        off = base + c * _CC
        pltpu.sync_copy(p0_hbm.at[pl.ds(off, _CC)], idx0_v)
        pltpu.sync_copy(p1_hbm.at[pl.ds(off, _CC)], idx1_v)
        pltpu.async_copy(ysw_hbm.at[idx0_v], rows0_v, sem).wait()
        pltpu.async_copy(ysw_hbm.at[idx1_v], rows1_v, sem).wait()

        def body(j, carry):
            for l in range(_D // _L):
                sl = pl.ds(l * _L, _L)
                rows0_v[j, sl] = rows0_v[j, sl] + rows1_v[j, sl]
            return carry

        lax.fori_loop(0, _CC, body, 0)
        pltpu.sync_copy(rows0_v, out_hbm.at[pl.ds(off, _CC)])


_sc_combine = functools.partial(
    pl.kernel,
    out_type=jax.ShapeDtypeStruct((_T, _D), jnp.float32),
    mesh=plsc.VectorSubcoreMesh(core_axis_name="c", subcore_axis_name="s"),
    scratch_types=[
        pltpu.VMEM((_CC,), jnp.int32),
        pltpu.VMEM((_CC,), jnp.int32),
        pltpu.VMEM((_CC, _D), jnp.float32),
        pltpu.VMEM((_CC, _D), jnp.float32),
        pltpu.SemaphoreType.DMA,
    ],
)(_sc_combine_body)


def kernel(x, Wr, gate, up, down):
    b, s, d = x.shape
    T = b * s
    xf = x.reshape(T, d)

    router_logits = xf @ Wr.T
    routing_weights = jax.nn.softmax(router_logits.astype(jnp.float32), axis=1)
    top_w, top_i = jax.lax.top_k(routing_weights, _K)

    expert_flat = top_i.reshape(-1).astype(jnp.int32)
    token_flat = (jnp.arange(_N, dtype=jnp.int32) // _K)
    onehot = (expert_flat[:, None]
              == jnp.arange(_E, dtype=jnp.int32)[None, :]).astype(jnp.int32)
    ranks = jnp.cumsum(onehot, axis=0) - onehot
    rank = jnp.sum(ranks * onehot, axis=1)
    counts = jnp.sum(onehot, axis=0)
    padded = ((counts + _TS - 1) // _TS) * _TS
    pstart = jnp.concatenate([jnp.zeros(1, padded.dtype), jnp.cumsum(padded)])
    p_flat = (pstart[expert_flat] + rank).astype(jnp.int32)
    row_token = jnp.zeros(_CAP, jnp.int32).at[p_flat].set(token_flat)
    w_row = jnp.zeros(_CAP, jnp.float32).at[p_flat].set(
        top_w.reshape(-1).astype(jnp.float32))
    pos = p_flat.reshape(T, _K)
    expert_of_tile = jnp.clip(
        jnp.searchsorted(pstart[1:], jnp.arange(_NT) * _TS, side="right"),
        0, _E - 1).astype(jnp.int32)

    xbf = xf.astype(jnp.bfloat16)
    xs = xbf[row_token]

    gate_bf = gate.astype(jnp.bfloat16)
    up_bf = up.astype(jnp.bfloat16)
    down_bf = down.astype(jnp.bfloat16)

    ysw = pl.pallas_call(
        _group_ffn_kernel,
        grid_spec=pltpu.PrefetchScalarGridSpec(
            num_scalar_prefetch=1,
            grid=(_NT,),
            in_specs=[
                pl.BlockSpec((_TS, _D), lambda j, eot: (j, 0)),
                pl.BlockSpec((_TS, 1), lambda j, eot: (j, 0)),
                pl.BlockSpec((1, _FF, _D), lambda j, eot: (eot[j], 0, 0)),
                pl.BlockSpec((1, _FF, _D), lambda j, eot: (eot[j], 0, 0)),
                pl.BlockSpec((1, _D, _FF), lambda j, eot: (eot[j], 0, 0)),
            ],
            out_specs=pl.BlockSpec((_TS, _D), lambda j, eot: (j, 0)),
        ),
        out_shape=jax.ShapeDtypeStruct((_CAP, _D), jnp.float32),
        compiler_params=pltpu.CompilerParams(
            dimension_semantics=("arbitrary",),
        ),
    )(expert_of_tile, xs, w_row[:, None], gate_bf, up_bf, down_bf)

    p0 = pos[:, 0]
    p1 = pos[:, 1]
    final = _sc_combine(ysw, p0, p1)

    return final.reshape(b, s, d), router_logits


# speedup vs baseline: 1.0212x; 1.0212x over previous
import functools

import jax
import jax.numpy as jnp
from jax import lax
from jax.experimental import pallas as pl
from jax.experimental.pallas import tpu as pltpu
from jax.experimental.pallas import tpu_sc as plsc

_B, _S, _D = 2, 4096, 1024
_E, _K, _FF = 8, 2, 4096
_T = _B * _S

_TS = 256
_N = _T * _K
_CAP = _N + _E * _TS
_NT = _CAP // _TS

_NC, _NS, _L = 2, 16, 16
_NW = _NC * _NS
_BPW = _T // _NW
_CC = 16
_NCH = _BPW // _CC


def _group_ffn_kernel(eot_ref, xs_ref, w_ref, gate_ref, up_ref, down_ref,
                      ys_ref):
    xt = xs_ref[...]
    g = gate_ref[0]
    u = up_ref[0]
    dn = down_ref[0]
    a = lax.dot_general(xt, g,
                        dimension_numbers=(((1,), (1,)), ((), ())),
                        preferred_element_type=jnp.float32)
    bb = lax.dot_general(xt, u,
                         dimension_numbers=(((1,), (1,)), ((), ())),
                         preferred_element_type=jnp.float32)
    h = (a * lax.logistic(a) * bb).astype(jnp.bfloat16)
    y = lax.dot_general(h, dn,
                        dimension_numbers=(((1,), (1,)), ((), ())),
                        preferred_element_type=jnp.float32)
    ys_ref[...] = y * w_ref[...]


def _sc_combine_body(ysw_hbm, p0_hbm, p1_hbm, out_hbm,
                     idx0_v, idx1_v, r0a, r1a, r0b, r1b,
                     sga, sgb, soa, sob):
    wid = lax.axis_index("s") * _NC + lax.axis_index("c")
    base = wid * _BPW
    pltpu.sync_copy(p0_hbm.at[pl.ds(base, _BPW)], idx0_v)
    pltpu.sync_copy(p1_hbm.at[pl.ds(base, _BPW)], idx1_v)

    bufs = ((r0a, r1a, sga, soa), (r0b, r1b, sgb, sob))

    def start_gather(c, r0, r1, sg):
        sl = pl.ds(c * _CC, _CC)
        h0 = pltpu.async_copy(ysw_hbm.at[idx0_v.at[sl]], r0, sg)
        h1 = pltpu.async_copy(ysw_hbm.at[idx1_v.at[sl]], r1, sg)
        return h0, h1

    ghandles = [None, None]
    ohandles = [None, None]
    ghandles[0] = start_gather(0, bufs[0][0], bufs[0][1], bufs[0][2])
    for c in range(_NCH):
        p = c & 1
        r0, r1, sg, so = bufs[p]
        if c + 1 < _NCH:
            q = 1 - p
            if ohandles[q] is not None:
                ohandles[q].wait()
            ghandles[q] = start_gather(c + 1, bufs[q][0], bufs[q][1],
                                       bufs[q][2])
        h0, h1 = ghandles[p]
        h0.wait()
        h1.wait()

        def body(j, carry):
            for l in range(_D // _L):
                sl = pl.ds(l * _L, _L)
                r0[j, sl] = r0[j, sl] + r1[j, sl]
            return carry

        lax.fori_loop(0, _CC, body, 0)
        ohandles[p] = pltpu.async_copy(
            r0, out_hbm.at[pl.ds(base + c * _CC, _CC)], so)
    ohandles[0].wait()
    ohandles[1].wait()


_sc_combine = functools.partial(
    pl.kernel,
    out_type=jax.ShapeDtypeStruct((_T, _D), jnp.float32),
    mesh=plsc.VectorSubcoreMesh(core_axis_name="c", subcore_axis_name="s"),
    scratch_types=[
        pltpu.VMEM((_BPW,), jnp.int32),
        pltpu.VMEM((_BPW,), jnp.int32),
        pltpu.VMEM((_CC, _D), jnp.float32),
        pltpu.VMEM((_CC, _D), jnp.float32),
        pltpu.VMEM((_CC, _D), jnp.float32),
        pltpu.VMEM((_CC, _D), jnp.float32),
        pltpu.SemaphoreType.DMA,
        pltpu.SemaphoreType.DMA,
        pltpu.SemaphoreType.DMA,
        pltpu.SemaphoreType.DMA,
    ],
)(_sc_combine_body)


def kernel(x, Wr, gate, up, down):
    b, s, d = x.shape
    T = b * s
    xf = x.reshape(T, d)

    router_logits = xf @ Wr.T
    routing_weights = jax.nn.softmax(router_logits.astype(jnp.float32), axis=1)
    top_w, top_i = jax.lax.top_k(routing_weights, _K)

    expert_flat = top_i.reshape(-1).astype(jnp.int32)
    token_flat = (jnp.arange(_N, dtype=jnp.int32) // _K)
    onehot = (expert_flat[None, :]
              == jnp.arange(_E, dtype=jnp.int32)[:, None]).astype(jnp.int32)
    ranks = jnp.cumsum(onehot, axis=1) - onehot
    rank = jnp.sum(ranks * onehot, axis=0)
    counts = jnp.sum(onehot, axis=1)
    padded = ((counts + _TS - 1) // _TS) * _TS
    pstart = jnp.concatenate([jnp.zeros(1, padded.dtype), jnp.cumsum(padded)])
    p_flat = (pstart[expert_flat] + rank).astype(jnp.int32)
    row_token = jnp.zeros(_CAP, jnp.int32).at[p_flat].set(token_flat)
    w_row = jnp.zeros(_CAP, jnp.float32).at[p_flat].set(
        top_w.reshape(-1).astype(jnp.float32))
    pos = p_flat.reshape(T, _K)
    expert_of_tile = jnp.clip(
        jnp.searchsorted(pstart[1:], jnp.arange(_NT) * _TS, side="right"),
        0, _E - 1).astype(jnp.int32)

    xbf = xf.astype(jnp.bfloat16)
    xs = xbf[row_token]

    gate_bf = gate.astype(jnp.bfloat16)
    up_bf = up.astype(jnp.bfloat16)
    down_bf = down.astype(jnp.bfloat16)

    ysw = pl.pallas_call(
        _group_ffn_kernel,
        grid_spec=pltpu.PrefetchScalarGridSpec(
            num_scalar_prefetch=1,
            grid=(_NT,),
            in_specs=[
                pl.BlockSpec((_TS, _D), lambda j, eot: (j, 0)),
                pl.BlockSpec((_TS, 1), lambda j, eot: (j, 0)),
                pl.BlockSpec((1, _FF, _D), lambda j, eot: (eot[j], 0, 0)),
                pl.BlockSpec((1, _FF, _D), lambda j, eot: (eot[j], 0, 0)),
                pl.BlockSpec((1, _D, _FF), lambda j, eot: (eot[j], 0, 0)),
            ],
            out_specs=pl.BlockSpec((_TS, _D), lambda j, eot: (j, 0)),
        ),
        out_shape=jax.ShapeDtypeStruct((_CAP, _D), jnp.float32),
        compiler_params=pltpu.CompilerParams(
            dimension_semantics=("arbitrary",),
        ),
    )(expert_of_tile, xs, w_row[:, None], gate_bf, up_bf, down_bf)

    p0 = pos[:, 0]
    p1 = pos[:, 1]
    final = _sc_combine(ysw, p0, p1)

    return final.reshape(b, s, d), router_logits


# speedup vs baseline: 1.0245x; 1.0033x over previous
import functools

import jax
import jax.numpy as jnp
from jax import lax
from jax.experimental import pallas as pl
from jax.experimental.pallas import tpu as pltpu
from jax.experimental.pallas import tpu_sc as plsc

_B, _S, _D = 2, 4096, 1024
_E, _K, _FF = 8, 2, 4096
_T = _B * _S

_TS = 256
_N = _T * _K
_CAP = _N + _E * _TS
_NT = _CAP // _TS

_NC, _NS, _L = 2, 16, 16
_NW = _NC * _NS
_BPW = _T // _NW
_CC = 16
_NCH = _BPW // _CC


def _group_ffn_kernel(eot_ref, xs_ref, w_ref, gate_ref, up_ref, down_ref,
                      ys_ref):
    xt = xs_ref[...]
    g = gate_ref[0]
    u = up_ref[0]
    dn = down_ref[0]
    a = lax.dot_general(xt, g,
                        dimension_numbers=(((1,), (1,)), ((), ())),
                        preferred_element_type=jnp.float32)
    bb = lax.dot_general(xt, u,
                         dimension_numbers=(((1,), (1,)), ((), ())),
                         preferred_element_type=jnp.float32)
    h = (a * lax.logistic(a) * bb).astype(jnp.bfloat16)
    y = lax.dot_general(h, dn,
                        dimension_numbers=(((1,), (1,)), ((), ())),
                        preferred_element_type=jnp.float32)
    ys_ref[...] = y * w_ref[...]


def _sc_combine_body(ysw_hbm, p0_hbm, p1_hbm, out_hbm,
                     idx0_v, idx1_v, r0a, r1a, r0b, r1b,
                     sga, sgb, soa, sob):
    wid = lax.axis_index("s") * _NC + lax.axis_index("c")
    base = wid * _BPW
    pltpu.sync_copy(p0_hbm.at[pl.ds(base, _BPW)], idx0_v)
    pltpu.sync_copy(p1_hbm.at[pl.ds(base, _BPW)], idx1_v)

    bufs = ((r0a, r1a, sga, soa), (r0b, r1b, sgb, sob))

    def start_gather(c, r0, r1, sg):
        sl = pl.ds(c * _CC, _CC)
        h0 = pltpu.async_copy(ysw_hbm.at[idx0_v.at[sl]], r0, sg)
        h1 = pltpu.async_copy(ysw_hbm.at[idx1_v.at[sl]], r1, sg)
        return h0, h1

    ghandles = [None, None]
    ohandles = [None, None]
    ghandles[0] = start_gather(0, bufs[0][0], bufs[0][1], bufs[0][2])
    for c in range(_NCH):
        p = c & 1
        r0, r1, sg, so = bufs[p]
        if c + 1 < _NCH:
            q = 1 - p
            if ohandles[q] is not None:
                ohandles[q].wait()
            ghandles[q] = start_gather(c + 1, bufs[q][0], bufs[q][1],
                                       bufs[q][2])
        h0, h1 = ghandles[p]
        h0.wait()
        h1.wait()

        def body(j, carry):
            for l in range(_D // _L):
                sl = pl.ds(l * _L, _L)
                r0[j, sl] = r0[j, sl] + r1[j, sl]
            return carry

        lax.fori_loop(0, _CC, body, 0)
        ohandles[p] = pltpu.async_copy(
            r0, out_hbm.at[pl.ds(base + c * _CC, _CC)], so)
    ohandles[0].wait()
    ohandles[1].wait()


_sc_combine = functools.partial(
    pl.kernel,
    out_type=jax.ShapeDtypeStruct((_T, _D), jnp.float32),
    mesh=plsc.VectorSubcoreMesh(core_axis_name="c", subcore_axis_name="s"),
    scratch_types=[
        pltpu.VMEM((_BPW,), jnp.int32),
        pltpu.VMEM((_BPW,), jnp.int32),
        pltpu.VMEM((_CC, _D), jnp.float32),
        pltpu.VMEM((_CC, _D), jnp.float32),
        pltpu.VMEM((_CC, _D), jnp.float32),
        pltpu.VMEM((_CC, _D), jnp.float32),
        pltpu.SemaphoreType.DMA,
        pltpu.SemaphoreType.DMA,
        pltpu.SemaphoreType.DMA,
        pltpu.SemaphoreType.DMA,
    ],
)(_sc_combine_body)


def kernel(x, Wr, gate, up, down):
    b, s, d = x.shape
    T = b * s
    xf = x.reshape(T, d)

    router_logits = xf @ Wr.T
    routing_weights = jax.nn.softmax(router_logits.astype(jnp.float32), axis=1)
    top_w, top_i = jax.lax.top_k(routing_weights, _K)

    expert_flat = top_i.reshape(-1).astype(jnp.int32)
    token_flat = (jnp.arange(_N, dtype=jnp.int32) // _K)
    _MB = 256
    nblk = _N // _MB
    onehot = (expert_flat[:, None]
              == jnp.arange(_E, dtype=jnp.int32)[None, :])
    oh_b = onehot.reshape(nblk, _MB, _E).astype(jnp.float32)
    tril = jnp.tril(jnp.ones((_MB, _MB), jnp.bfloat16), -1)
    within = jax.lax.dot_general(
        oh_b.astype(jnp.bfloat16), tril,
        dimension_numbers=(((1,), (1,)), ((), ())),
        preferred_element_type=jnp.float32)
    within = jnp.swapaxes(within, 1, 2)
    block_tot = jnp.sum(oh_b, axis=1)
    offs = jnp.cumsum(block_tot, axis=0) - block_tot
    ranks_f = within + offs[:, None, :]
    rank = jnp.sum(ranks_f.reshape(_N, _E) * onehot,
                   axis=1).astype(jnp.int32)
    counts = jnp.sum(block_tot, axis=0).astype(jnp.int32)
    padded = ((counts + _TS - 1) // _TS) * _TS
    pstart = jnp.concatenate([jnp.zeros(1, padded.dtype), jnp.cumsum(padded)])
    p_flat = (pstart[expert_flat] + rank).astype(jnp.int32)
    row_token = jnp.zeros(_CAP, jnp.int32).at[p_flat].set(token_flat)
    w_row = jnp.zeros(_CAP, jnp.float32).at[p_flat].set(
        top_w.reshape(-1).astype(jnp.float32))
    pos = p_flat.reshape(T, _K)
    expert_of_tile = jnp.clip(
        jnp.searchsorted(pstart[1:], jnp.arange(_NT) * _TS, side="right"),
        0, _E - 1).astype(jnp.int32)

    xbf = xf.astype(jnp.bfloat16)
    xs = xbf[row_token]

    gate_bf = gate.astype(jnp.bfloat16)
    up_bf = up.astype(jnp.bfloat16)
    down_bf = down.astype(jnp.bfloat16)

    ysw = pl.pallas_call(
        _group_ffn_kernel,
        grid_spec=pltpu.PrefetchScalarGridSpec(
            num_scalar_prefetch=1,
            grid=(_NT,),
            in_specs=[
                pl.BlockSpec((_TS, _D), lambda j, eot: (j, 0)),
                pl.BlockSpec((_TS, 1), lambda j, eot: (j, 0)),
                pl.BlockSpec((1, _FF, _D), lambda j, eot: (eot[j], 0, 0)),
                pl.BlockSpec((1, _FF, _D), lambda j, eot: (eot[j], 0, 0)),
                pl.BlockSpec((1, _D, _FF), lambda j, eot: (eot[j], 0, 0)),
            ],
            out_specs=pl.BlockSpec((_TS, _D), lambda j, eot: (j, 0)),
        ),
        out_shape=jax.ShapeDtypeStruct((_CAP, _D), jnp.float32),
        compiler_params=pltpu.CompilerParams(
            dimension_semantics=("arbitrary",),
        ),
    )(expert_of_tile, xs, w_row[:, None], gate_bf, up_bf, down_bf)

    p0 = pos[:, 0]
    p1 = pos[:, 1]
    final = _sc_combine(ysw, p0, p1)

    return final.reshape(b, s, d), router_logits


# speedup vs baseline: 1.0353x; 1.0105x over previous
import functools

import jax
import jax.numpy as jnp
from jax import lax
from jax.experimental import pallas as pl
from jax.experimental.pallas import tpu as pltpu
from jax.experimental.pallas import tpu_sc as plsc

_B, _S, _D = 2, 4096, 1024
_E, _K, _FF = 8, 2, 4096
_T = _B * _S

_TS = 256
_N = _T * _K
_CAP = _N + _E * _TS
_NT = _CAP // _TS

_NC, _NS, _L = 2, 16, 16
_NW = _NC * _NS
_BPW = _T // _NW
_CC = 16
_NCH = _BPW // _CC


def _group_ffn_kernel(eot_ref, xs_ref, w_ref, gate_ref, up_ref, down_ref,
                      ys_ref):
    xt = xs_ref[...]
    g = gate_ref[0]
    u = up_ref[0]
    dn = down_ref[0]
    a = lax.dot_general(xt, g,
                        dimension_numbers=(((1,), (1,)), ((), ())),
                        preferred_element_type=jnp.float32)
    bb = lax.dot_general(xt, u,
                         dimension_numbers=(((1,), (1,)), ((), ())),
                         preferred_element_type=jnp.float32)
    h = (a * lax.logistic(a) * bb).astype(jnp.bfloat16)
    y = lax.dot_general(h, dn,
                        dimension_numbers=(((1,), (1,)), ((), ())),
                        preferred_element_type=jnp.float32)
    ys_ref[...] = y * w_ref[...]


def _sc_combine_body(ysw_hbm, p0_hbm, p1_hbm, out_hbm,
                     idx0_v, idx1_v, r0a, r1a, r0b, r1b,
                     sga, sgb, soa, sob):
    wid = lax.axis_index("s") * _NC + lax.axis_index("c")
    base = wid * _BPW
    pltpu.sync_copy(p0_hbm.at[pl.ds(base, _BPW)], idx0_v)
    pltpu.sync_copy(p1_hbm.at[pl.ds(base, _BPW)], idx1_v)

    bufs = ((r0a, r1a, sga, soa), (r0b, r1b, sgb, sob))

    def start_gather(c, r0, r1, sg):
        sl = pl.ds(c * _CC, _CC)
        h0 = pltpu.async_copy(ysw_hbm.at[idx0_v.at[sl]], r0, sg)
        h1 = pltpu.async_copy(ysw_hbm.at[idx1_v.at[sl]], r1, sg)
        return h0, h1

    ghandles = [None, None]
    ohandles = [None, None]
    ghandles[0] = start_gather(0, bufs[0][0], bufs[0][1], bufs[0][2])
    for c in range(_NCH):
        p = c & 1
        r0, r1, sg, so = bufs[p]
        if c + 1 < _NCH:
            q = 1 - p
            if ohandles[q] is not None:
                ohandles[q].wait()
            ghandles[q] = start_gather(c + 1, bufs[q][0], bufs[q][1],
                                       bufs[q][2])
        h0, h1 = ghandles[p]
        h0.wait()
        h1.wait()

        def body(j, carry):
            for l in range(_D // _L):
                sl = pl.ds(l * _L, _L)
                r0[j, sl] = r0[j, sl] + r1[j, sl]
            return carry

        lax.fori_loop(0, _CC, body, 0)
        ohandles[p] = pltpu.async_copy(
            r0, out_hbm.at[pl.ds(base + c * _CC, _CC)], so)
    ohandles[0].wait()
    ohandles[1].wait()


_sc_combine = functools.partial(
    pl.kernel,
    out_type=jax.ShapeDtypeStruct((_T, _D), jnp.float32),
    mesh=plsc.VectorSubcoreMesh(core_axis_name="c", subcore_axis_name="s"),
    scratch_types=[
        pltpu.VMEM((_BPW,), jnp.int32),
        pltpu.VMEM((_BPW,), jnp.int32),
        pltpu.VMEM((_CC, _D), jnp.float32),
        pltpu.VMEM((_CC, _D), jnp.float32),
        pltpu.VMEM((_CC, _D), jnp.float32),
        pltpu.VMEM((_CC, _D), jnp.float32),
        pltpu.SemaphoreType.DMA,
        pltpu.SemaphoreType.DMA,
        pltpu.SemaphoreType.DMA,
        pltpu.SemaphoreType.DMA,
    ],
)(_sc_combine_body)


def kernel(x, Wr, gate, up, down):
    b, s, d = x.shape
    T = b * s
    xf = x.reshape(T, d)

    router_logits = xf @ Wr.T
    routing_weights = jax.nn.softmax(router_logits.astype(jnp.float32), axis=1)
    i1 = jnp.argmax(routing_weights, axis=1)
    m1 = jnp.max(routing_weights, axis=1)
    lane = jnp.arange(_E, dtype=jnp.int32)[None, :]
    rw2 = jnp.where(lane == i1[:, None], -jnp.inf, routing_weights)
    i2 = jnp.argmax(rw2, axis=1)
    m2 = jnp.max(rw2, axis=1)
    top_w = jnp.stack([m1, m2], axis=1)
    top_i = jnp.stack([i1, i2], axis=1).astype(jnp.int32)

    expert_flat = top_i.reshape(-1).astype(jnp.int32)
    token_flat = (jnp.arange(_N, dtype=jnp.int32) // _K)
    _MB = 256
    nblk = _N // _MB
    onehot = (expert_flat[:, None]
              == jnp.arange(_E, dtype=jnp.int32)[None, :])
    oh_b = onehot.reshape(nblk, _MB, _E).astype(jnp.float32)
    tril = jnp.tril(jnp.ones((_MB, _MB), jnp.bfloat16), -1)
    within = jax.lax.dot_general(
        oh_b.astype(jnp.bfloat16), tril,
        dimension_numbers=(((1,), (1,)), ((), ())),
        preferred_element_type=jnp.float32)
    within = jnp.swapaxes(within, 1, 2)
    block_tot = jnp.sum(oh_b, axis=1)
    offs = jnp.cumsum(block_tot, axis=0) - block_tot
    ranks_f = within + offs[:, None, :]
    rank = jnp.sum(ranks_f.reshape(_N, _E) * onehot,
                   axis=1).astype(jnp.int32)
    counts = jnp.sum(block_tot, axis=0).astype(jnp.int32)
    padded = ((counts + _TS - 1) // _TS) * _TS
    pstart = jnp.concatenate([jnp.zeros(1, padded.dtype), jnp.cumsum(padded)])
    p_flat = (pstart[expert_flat] + rank).astype(jnp.int32)
    row_token = jnp.zeros(_CAP, jnp.int32).at[p_flat].set(token_flat)
    w_row = jnp.zeros(_CAP, jnp.float32).at[p_flat].set(
        top_w.reshape(-1).astype(jnp.float32))
    pos = p_flat.reshape(T, _K)
    expert_of_tile = jnp.clip(
        jnp.searchsorted(pstart[1:], jnp.arange(_NT) * _TS, side="right"),
        0, _E - 1).astype(jnp.int32)

    xbf = xf.astype(jnp.bfloat16)
    xs = xbf[row_token]

    gate_bf = gate.astype(jnp.bfloat16)
    up_bf = up.astype(jnp.bfloat16)
    down_bf = down.astype(jnp.bfloat16)

    ysw = pl.pallas_call(
        _group_ffn_kernel,
        grid_spec=pltpu.PrefetchScalarGridSpec(
            num_scalar_prefetch=1,
            grid=(_NT,),
            in_specs=[
                pl.BlockSpec((_TS, _D), lambda j, eot: (j, 0)),
                pl.BlockSpec((_TS, 1), lambda j, eot: (j, 0)),
                pl.BlockSpec((1, _FF, _D), lambda j, eot: (eot[j], 0, 0)),
                pl.BlockSpec((1, _FF, _D), lambda j, eot: (eot[j], 0, 0)),
                pl.BlockSpec((1, _D, _FF), lambda j, eot: (eot[j], 0, 0)),
            ],
            out_specs=pl.BlockSpec((_TS, _D), lambda j, eot: (j, 0)),
        ),
        out_shape=jax.ShapeDtypeStruct((_CAP, _D), jnp.float32),
        compiler_params=pltpu.CompilerParams(
            dimension_semantics=("arbitrary",),
        ),
    )(expert_of_tile, xs, w_row[:, None], gate_bf, up_bf, down_bf)

    p0 = pos[:, 0]
    p1 = pos[:, 1]
    final = _sc_combine(ysw, p0, p1)

    return final.reshape(b, s, d), router_logits
